# baseline (speedup 1.0000x reference)
import functools

import jax
import jax.numpy as jnp
from jax import lax
from jax.experimental import pallas as pl
from jax.experimental.pallas import tpu as pltpu
from jax.experimental.pallas import tpu_sc as plsc

B = 4
S = 8192
D = 1024

S_SC = 4096
S_TC = S - S_SC

NC = 2
NS = 16
NW = NC * NS
S_PER_W = S_SC // NW
K = 16
NCHUNK = S_PER_W // K
NSTEP = NCHUNK * B
LANES = 16


def _body(x_hbm, pe_hbm, out_hbm,
          pe0, pe1, xin0, xin1, xout0, xout1,
          pe_sem0, pe_sem1, ld_sem0, ld_sem1, st_sem0, st_sem1):
    wid = lax.axis_index("s") * NC + lax.axis_index("c")
    base = wid * S_PER_W

    pe_bufs = [pe0, pe1]
    xins = [xin0, xin1]
    xouts = [xout0, xout1]
    pe_sems = [pe_sem0, pe_sem1]
    ld_sems = [ld_sem0, ld_sem1]
    st_sems = [st_sem0, st_sem1]

    def pe_load(c):
        return pltpu.async_copy(
            pe_hbm.at[pl.ds(base + c * K, K)], pe_bufs[c % 2], pe_sems[c % 2])

    def x_load(t):
        c, b = divmod(t, B)
        row = b * S + base + c * K
        return pltpu.async_copy(
            x_hbm.at[pl.ds(row, K)], xins[t % 2], ld_sems[t % 2])

    def x_store(t):
        c, b = divmod(t, B)
        row = b * S_SC + base + c * K
        return pltpu.async_copy(
            xouts[t % 2], out_hbm.at[pl.ds(row, K)], st_sems[t % 2])

    pe_h = [pe_load(0), None]
    ld_h = [x_load(0), None]
    st_h = [None, None]

    for t in range(NSTEP):
        c, b = divmod(t, B)
        p = t % 2
        if t + 1 < NSTEP:
            ld_h[(t + 1) % 2] = x_load(t + 1)
        if b == 0 and c + 1 < NCHUNK:
            pe_h[(c + 1) % 2] = pe_load(c + 1)
        ld_h[p].wait()
        if b == 0:
            pe_h[c % 2].wait()
        if st_h[p] is not None:
            st_h[p].wait()

        xin, xout, pe_buf = xins[p], xouts[p], pe_bufs[c % 2]

        def add_body(i, carry):
            sl = pl.ds(i * LANES, LANES)
            for r in range(K):
                xout[r, sl] = xin[r, sl] + pe_buf[r, sl]
            return carry

        lax.fori_loop(0, D // LANES, add_body, 0)
        st_h[p] = x_store(t)

    st_h[(NSTEP - 2) % 2].wait()
    st_h[(NSTEP - 1) % 2].wait()


_mesh = plsc.VectorSubcoreMesh(core_axis_name="c", subcore_axis_name="s")

_sc_add = pl.kernel(
    _body,
    mesh=_mesh,
    out_type=jax.ShapeDtypeStruct((B * S_SC, D), jnp.float32),
    scratch_types=[
        pltpu.VMEM((K, D), jnp.float32),
        pltpu.VMEM((K, D), jnp.float32),
        pltpu.VMEM((K, D), jnp.float32),
        pltpu.VMEM((K, D), jnp.float32),
        pltpu.VMEM((K, D), jnp.float32),
        pltpu.VMEM((K, D), jnp.float32),
        pltpu.SemaphoreType.DMA,
        pltpu.SemaphoreType.DMA,
        pltpu.SemaphoreType.DMA,
        pltpu.SemaphoreType.DMA,
        pltpu.SemaphoreType.DMA,
        pltpu.SemaphoreType.DMA,
    ],
    compiler_params=pltpu.CompilerParams(use_tc_tiling_on_sc=True),
)


BS_TC = 512


def _tc_body(x_ref, pe_ref, out_ref):
    out_ref[0] = x_ref[0] + pe_ref[...]


_SOFF = S_SC // BS_TC

_tc_add = pl.pallas_call(
    _tc_body,
    grid=(S_TC // BS_TC, B),
    in_specs=[
        pl.BlockSpec((1, BS_TC, D), lambda s, b: (b, s + _SOFF, 0)),
        pl.BlockSpec((BS_TC, D), lambda s, b: (s + _SOFF, 0)),
    ],
    out_specs=pl.BlockSpec((1, BS_TC, D), lambda s, b: (b, s, 0)),
    out_shape=jax.ShapeDtypeStruct((B, S_TC, D), jnp.float32),
)


def kernel(x, pos_embedding):
    x2d = x.reshape(B * S, D)
    sc_half = _sc_add(x2d, pos_embedding).reshape(B, S_SC, D)
    tc_half = _tc_add(x, pos_embedding)
    return jnp.concatenate([sc_half, tc_half], axis=1)


# speedup vs baseline: 1.1459x; 1.1459x over previous
import functools

import jax
import jax.numpy as jnp
from jax import lax
from jax.experimental import pallas as pl
from jax.experimental.pallas import tpu as pltpu
from jax.experimental.pallas import tpu_sc as plsc

B = 4
S = 8192
D = 1024

NC = 2
NS = 16
NW = NC * NS
S_PER_W = S // NW
K = 16
NCHUNK = S_PER_W // K
NSTEP = NCHUNK * B
LANES = 16


def _body(x_hbm, pe_hbm, out_hbm,
          pe0, pe1, xin0, xin1, xout0, xout1,
          pe_sem0, pe_sem1, ld_sem0, ld_sem1, st_sem0, st_sem1):
    wid = lax.axis_index("s") * NC + lax.axis_index("c")
    base = wid * S_PER_W

    pe_bufs = [pe0, pe1]
    xins = [xin0, xin1]
    xouts = [xout0, xout1]
    pe_sems = [pe_sem0, pe_sem1]
    ld_sems = [ld_sem0, ld_sem1]
    st_sems = [st_sem0, st_sem1]

    def pe_load(c):
        return pltpu.async_copy(
            pe_hbm.at[pl.ds(base + c * K, K)], pe_bufs[c % 2], pe_sems[c % 2])

    def x_load(t):
        c, b = divmod(t, B)
        row = b * S + base + c * K
        return pltpu.async_copy(
            x_hbm.at[pl.ds(row, K)], xins[t % 2], ld_sems[t % 2])

    def x_store(t):
        c, b = divmod(t, B)
        row = b * S + base + c * K
        return pltpu.async_copy(
            xouts[t % 2], out_hbm.at[pl.ds(row, K)], st_sems[t % 2])

    pe_h = [pe_load(0), None]
    ld_h = [x_load(0), None]
    st_h = [None, None]

    for t in range(NSTEP):
        c, b = divmod(t, B)
        p = t % 2
        if t + 1 < NSTEP:
            ld_h[(t + 1) % 2] = x_load(t + 1)
        if b == 0 and c + 1 < NCHUNK:
            pe_h[(c + 1) % 2] = pe_load(c + 1)
        ld_h[p].wait()
        if b == 0:
            pe_h[c % 2].wait()
        if st_h[p] is not None:
            st_h[p].wait()

        xin, xout, pe_buf = xins[p], xouts[p], pe_bufs[c % 2]

        def add_body(i, carry):
            sl = pl.ds(i * LANES, LANES)
            for r in range(K):
                xout[r, sl] = xin[r, sl] + pe_buf[r, sl]
            return carry

        lax.fori_loop(0, D // LANES, add_body, 0)
        st_h[p] = x_store(t)

    st_h[(NSTEP - 2) % 2].wait()
    st_h[(NSTEP - 1) % 2].wait()


_mesh = plsc.VectorSubcoreMesh(core_axis_name="c", subcore_axis_name="s")

_sc_add = pl.kernel(
    _body,
    mesh=_mesh,
    out_type=jax.ShapeDtypeStruct((B * S, D), jnp.float32),
    scratch_types=[
        pltpu.VMEM((K, D), jnp.float32),
        pltpu.VMEM((K, D), jnp.float32),
        pltpu.VMEM((K, D), jnp.float32),
        pltpu.VMEM((K, D), jnp.float32),
        pltpu.VMEM((K, D), jnp.float32),
        pltpu.VMEM((K, D), jnp.float32),
        pltpu.SemaphoreType.DMA,
        pltpu.SemaphoreType.DMA,
        pltpu.SemaphoreType.DMA,
        pltpu.SemaphoreType.DMA,
        pltpu.SemaphoreType.DMA,
        pltpu.SemaphoreType.DMA,
    ],
    compiler_params=pltpu.CompilerParams(use_tc_tiling_on_sc=True),
)


def kernel(x, pos_embedding):
    out2d = _sc_add(x.reshape(B * S, D), pos_embedding)
    return out2d.reshape(x.shape)


# speedup vs baseline: 1.5341x; 1.3388x over previous
import functools

import jax
import jax.numpy as jnp
from jax import lax
from jax.experimental import pallas as pl
from jax.experimental.pallas import tpu as pltpu
from jax.experimental.pallas import tpu_sc as plsc

B = 4
S = 8192
D = 1024

NC = 2
NS = 16
NW = NC * NS
S_PER_W = S // NW
K = 8
NCHUNK = S_PER_W // K
LANES = 16


def _body(x_hbm, pe_hbm, out_hbm,
          pe0, pe1,
          xb00, xb01, xb10, xb11, xb20, xb21, xb30, xb31,
          pe_sem0, pe_sem1, ld_sem0, ld_sem1, st_sem0, st_sem1):
    wid = lax.axis_index("s") * NC + lax.axis_index("c")
    base = wid * S_PER_W

    pe_bufs = [pe0, pe1]
    xbufs = [[xb00, xb01], [xb10, xb11], [xb20, xb21], [xb30, xb31]]
    pe_sems = [pe_sem0, pe_sem1]
    ld_sems = [ld_sem0, ld_sem1]
    st_sems = [st_sem0, st_sem1]

    def pe_load(c):
        return pltpu.async_copy(
            pe_hbm.at[pl.ds(base + c * K, K)], pe_bufs[c % 2], pe_sems[c % 2])

    def x_load(b, c):
        row = b * S + base + c * K
        return pltpu.async_copy(
            x_hbm.at[pl.ds(row, K)], xbufs[b][c % 2], ld_sems[c % 2])

    def x_store(b, c):
        row = b * S + base + c * K
        return pltpu.async_copy(
            xbufs[b][c % 2], out_hbm.at[pl.ds(row, K)], st_sems[c % 2])

    pe_h = [pe_load(0), None]
    ld_h = [[x_load(b, 0) for b in range(B)], None]
    st_h = [None, None]

    for c in range(NCHUNK):
        p = c % 2
        q = (c + 1) % 2
        if c + 1 < NCHUNK:
            if st_h[q] is not None:
                for h in st_h[q]:
                    h.wait()
                st_h[q] = None
            pe_h[q] = pe_load(c + 1)
            ld_h[q] = [x_load(b, c + 1) for b in range(B)]
        for h in ld_h[p]:
            h.wait()
        pe_h[p].wait()

        pe_buf = pe_bufs[p]
        bufs = [xbufs[b][p] for b in range(B)]

        @plsc.parallel_loop(0, D // LANES, 1, unroll=2)
        def _add(i):
            sl = pl.ds(i * LANES, LANES)
            for r in range(K):
                v = pe_buf[r, sl]
                for b in range(B):
                    plsc.addupdate(bufs[b].at[r, sl], v)

        st_h[p] = [x_store(b, c) for b in range(B)]

    for hs in st_h:
        if hs is not None:
            for h in hs:
                h.wait()


_mesh = plsc.VectorSubcoreMesh(core_axis_name="c", subcore_axis_name="s")

_sc_add = pl.kernel(
    _body,
    mesh=_mesh,
    out_type=jax.ShapeDtypeStruct((B * S, D), jnp.float32),
    scratch_types=(
        [pltpu.VMEM((K, D), jnp.float32) for _ in range(10)]
        + [pltpu.SemaphoreType.DMA for _ in range(6)]
    ),
    compiler_params=pltpu.CompilerParams(use_tc_tiling_on_sc=True),
)


def kernel(x, pos_embedding):
    out2d = _sc_add(x.reshape(B * S, D), pos_embedding)
    return out2d.reshape(x.shape)
